# Initial kernel scaffold; baseline (speedup 1.0000x reference)
import functools

import jax
import jax.numpy as jnp
from jax import lax
from jax.experimental import pallas as pl
from jax.experimental.pallas import tpu as pltpu
from jax.experimental.pallas import tpu_sc as plsc

N, E, D = 10000, 320000, 128
NC, NS, L = 2, 16, 16
NW = NC * NS
EPT = E // NW
CHUNK = 80
NCHUNK = EPT // CHUNK
RPT = N // NS
ZROWS = 125


def _sc_scatter_add(x, src, dst):
    mesh = plsc.VectorSubcoreMesh(
        core_axis_name="c", subcore_axis_name="s", num_cores=NC, num_subcores=NS
    )

    @functools.partial(
        pl.kernel,
        out_type=jax.ShapeDtypeStruct((NC * N, D), jnp.float32),
        mesh=mesh,
        scratch_types=[
            pltpu.VMEM((CHUNK,), jnp.int32),
            pltpu.VMEM((CHUNK,), jnp.int32),
            pltpu.VMEM((CHUNK, D), jnp.float32),
            pltpu.VMEM((ZROWS, D), jnp.float32),
            pltpu.VMEM_SHARED((N, D), jnp.float32),
            pltpu.SemaphoreType.DMA,
        ],
    )
    def k(x_hbm, src_hbm, dst_hbm, parts_hbm, src_idx, dst_idx, rows, zbuf, agg, sem):
        cid = lax.axis_index("c")
        sid = lax.axis_index("s")
        wid = sid * NC + cid

        def zrow(i, _):
            def zcol(c, _):
                zbuf[i, pl.ds(c * L, L)] = jnp.zeros((L,), jnp.float32)
                return 0
            return lax.fori_loop(0, D // L, zcol, 0)
        lax.fori_loop(0, ZROWS, zrow, 0)

        r0 = sid * RPT
        for j in range(RPT // ZROWS):
            pltpu.sync_copy(zbuf, agg.at[pl.ds(r0 + j * ZROWS, ZROWS)])
        plsc.subcore_barrier()

        base = wid * EPT

        def step(i, _):
            off = base + i * CHUNK
            pltpu.sync_copy(src_hbm.at[pl.ds(off, CHUNK)], src_idx)
            pltpu.sync_copy(dst_hbm.at[pl.ds(off, CHUNK)], dst_idx)
            pltpu.async_copy(x_hbm.at[src_idx], rows, sem).wait()
            pltpu.sync_copy(rows, agg.at[dst_idx], add=True)
            return 0

        lax.fori_loop(0, NCHUNK, step, 0)
        plsc.subcore_barrier()

        pltpu.sync_copy(agg.at[pl.ds(r0, RPT)], parts_hbm.at[pl.ds(cid * N + r0, RPT)])

    return k(x, src, dst)


_BLK = 400


def _mlp_body(p0_ref, p1_ref, x_ref, w1_ref, b1_ref, w2_ref, b2_ref, o_ref):
    s = p0_ref[...] + p1_ref[...] + x_ref[...]
    h = jnp.maximum(
        jnp.dot(s, w1_ref[...], preferred_element_type=jnp.float32) + b1_ref[...], 0.0
    )
    o_ref[...] = jnp.dot(h, w2_ref[...], preferred_element_type=jnp.float32) + b2_ref[...]


def _mlp(p0, p1, x, W1, b1, W2, b2):
    return pl.pallas_call(
        _mlp_body,
        grid=(N // _BLK,),
        in_specs=[
            pl.BlockSpec((_BLK, D), lambda i: (i, 0)),
            pl.BlockSpec((_BLK, D), lambda i: (i, 0)),
            pl.BlockSpec((_BLK, D), lambda i: (i, 0)),
            pl.BlockSpec((D, D), lambda i: (0, 0)),
            pl.BlockSpec((1, D), lambda i: (0, 0)),
            pl.BlockSpec((D, D), lambda i: (0, 0)),
            pl.BlockSpec((1, D), lambda i: (0, 0)),
        ],
        out_specs=pl.BlockSpec((_BLK, D), lambda i: (i, 0)),
        out_shape=jax.ShapeDtypeStruct((N, D), jnp.float32),
    )(p0, p1, x, W1, b1.reshape(1, D), W2, b2.reshape(1, D))


@jax.jit
def kernel(x, edge_index, W1, b1, W2, b2):
    src = edge_index[0]
    dst = edge_index[1]
    parts = _sc_scatter_add(x, src, dst)
    return _mlp(parts[:N], parts[N:], x, W1, b1, W2, b2)


# speedup vs baseline: 4.8313x; 4.8313x over previous
import functools

import jax
import jax.numpy as jnp
from jax import lax
from jax.experimental import pallas as pl
from jax.experimental.pallas import tpu as pltpu
from jax.experimental.pallas import tpu_sc as plsc

N, E, D = 10000, 320000, 128
NP = 10240
NC, NS, L = 2, 16, 16
NW = NC * NS
EPT = E // NW
CHUNK = 80
NCHUNK = EPT // CHUNK
RPT = NP // NS
ZROWS = 128


def _sc_scatter_add(x, src, dst):
    mesh = plsc.VectorSubcoreMesh(
        core_axis_name="c", subcore_axis_name="s", num_cores=NC, num_subcores=NS
    )

    @functools.partial(
        pl.kernel,
        out_type=jax.ShapeDtypeStruct((NC * NP, D), jnp.float32),
        mesh=mesh,
        scratch_types=[
            pltpu.VMEM((CHUNK,), jnp.int32),
            pltpu.VMEM((CHUNK,), jnp.int32),
            pltpu.VMEM((CHUNK, D), jnp.float32),
            pltpu.VMEM((ZROWS, D), jnp.float32),
            pltpu.VMEM_SHARED((NP, D), jnp.float32),
            pltpu.SemaphoreType.DMA,
        ],
    )
    def k(x_hbm, src_hbm, dst_hbm, parts_hbm, src_idx, dst_idx, rows, zbuf, agg, sem):
        cid = lax.axis_index("c")
        sid = lax.axis_index("s")
        wid = sid * NC + cid

        def zrow(i, _):
            def zcol(c, _):
                zbuf[i, pl.ds(c * L, L)] = jnp.zeros((L,), jnp.float32)
                return 0
            return lax.fori_loop(0, D // L, zcol, 0)
        lax.fori_loop(0, ZROWS, zrow, 0)

        r0 = sid * RPT
        for j in range(RPT // ZROWS):
            pltpu.sync_copy(zbuf, agg.at[pl.ds(r0 + j * ZROWS, ZROWS)])
        plsc.subcore_barrier()

        base = wid * EPT

        def step(i, _):
            off = base + i * CHUNK
            pltpu.sync_copy(src_hbm.at[pl.ds(off, CHUNK)], src_idx)
            pltpu.sync_copy(dst_hbm.at[pl.ds(off, CHUNK)], dst_idx)
            pltpu.async_copy(x_hbm.at[src_idx], rows, sem).wait()
            pltpu.sync_copy(rows, agg.at[dst_idx], add=True)
            return 0

        lax.fori_loop(0, NCHUNK, step, 0)
        plsc.subcore_barrier()

        pltpu.sync_copy(agg.at[pl.ds(r0, RPT)], parts_hbm.at[pl.ds(cid * NP + r0, RPT)])

    return k(x, src, dst)


_BLK = 400


def _mlp_body(p0_ref, p1_ref, x_ref, w1_ref, b1_ref, w2_ref, b2_ref, o_ref):
    s = p0_ref[...] + p1_ref[...] + x_ref[...]
    h = jnp.maximum(
        jnp.dot(s, w1_ref[...], preferred_element_type=jnp.float32) + b1_ref[...], 0.0
    )
    o_ref[...] = jnp.dot(h, w2_ref[...], preferred_element_type=jnp.float32) + b2_ref[...]


def _mlp(p0, p1, x, W1, b1, W2, b2):
    return pl.pallas_call(
        _mlp_body,
        grid=(N // _BLK,),
        in_specs=[
            pl.BlockSpec((_BLK, D), lambda i: (i, 0)),
            pl.BlockSpec((_BLK, D), lambda i: (i, 0)),
            pl.BlockSpec((_BLK, D), lambda i: (i, 0)),
            pl.BlockSpec((D, D), lambda i: (0, 0)),
            pl.BlockSpec((1, D), lambda i: (0, 0)),
            pl.BlockSpec((D, D), lambda i: (0, 0)),
            pl.BlockSpec((1, D), lambda i: (0, 0)),
        ],
        out_specs=pl.BlockSpec((_BLK, D), lambda i: (i, 0)),
        out_shape=jax.ShapeDtypeStruct((N, D), jnp.float32),
    )(p0, p1, x, W1, b1.reshape(1, D), W2, b2.reshape(1, D))


@jax.jit
def kernel(x, edge_index, W1, b1, W2, b2):
    src = edge_index[0]
    dst = edge_index[1]
    parts = _sc_scatter_add(x, src, dst)
    return _mlp(parts[:N], parts[NP:NP + N], x, W1, b1, W2, b2)
